# Initial kernel scaffold; baseline (speedup 1.0000x reference)
import jax
import jax.numpy as jnp
from jax.experimental import pallas as pl


def kernel(x, w1, w2):
    raise NotImplementedError("write your pallas kernel here")


# speedup vs baseline: 1.8032x; 1.8032x over previous
import functools

import jax
import jax.numpy as jnp
from jax.experimental import pallas as pl
from jax.experimental.pallas import tpu as pltpu


def _ca_kernel(x_ref, w1t_ref, w2t_ref, o_ref, *, inv_hw):
    x = x_ref[...]
    pooled = jnp.sum(x, axis=-1) * inv_hw
    h = jnp.dot(pooled, w1t_ref[...], preferred_element_type=jnp.float32)
    h = jnp.maximum(h, 0.0)
    s = jnp.dot(h, w2t_ref[...], preferred_element_type=jnp.float32)
    s = jax.nn.sigmoid(s)
    o_ref[...] = x * s[:, :, None]


def kernel(x, w1, w2):
    N, C, H, W = x.shape
    Cr = w1.shape[0]
    HW = H * W

    x3 = x.reshape(N, C, HW)
    w1t = w1.T
    w2t = w2.T

    bn = 2
    assert N % bn == 0

    out = pl.pallas_call(
        functools.partial(_ca_kernel, inv_hw=1.0 / HW),
        out_shape=jax.ShapeDtypeStruct((N, C, HW), x.dtype),
        grid=(N // bn,),
        in_specs=[
            pl.BlockSpec((bn, C, HW), lambda n: (n, 0, 0)),
            pl.BlockSpec((C, Cr), lambda n: (0, 0)),
            pl.BlockSpec((Cr, C), lambda n: (0, 0)),
        ],
        out_specs=pl.BlockSpec((bn, C, HW), lambda n: (n, 0, 0)),
        compiler_params=pltpu.CompilerParams(
            dimension_semantics=("parallel",)),
    )(x3, w1t, w2t)

    return out.reshape(N, C, H, W)
